# Initial kernel scaffold; baseline (speedup 1.0000x reference)
import jax
import jax.numpy as jnp
from jax.experimental import pallas as pl


def kernel(x, edge_index, edge_weight, W1_0, b1_0, W2_0, W3_0, b3_0, W1_1, b1_1, W2_1, W3_1, b3_1, pc_W, pc_b):
    raise NotImplementedError("write your pallas kernel here")


# speedup vs baseline: 1.1174x; 1.1174x over previous
import jax
import jax.numpy as jnp
from jax import lax
from jax.experimental import pallas as pl

N = 10000
D = 256
E = 160000

HALF = 128
RB = 1000
GR = N // RB
NC = 2


def _lin_body(x_ref, w1_ref, b1_ref, w2_ref, w3_ref, b3_ref, a_ref, bc_ref):
    xb = x_ref[...]
    a = jnp.dot(xb, w1_ref[...], preferred_element_type=jnp.float32)
    a_ref[...] = a + b1_ref[...]
    b = jnp.dot(xb, w2_ref[...], preferred_element_type=jnp.float32)
    cc = jnp.dot(xb, w3_ref[...], preferred_element_type=jnp.float32)
    bc_ref[...] = jnp.concatenate([b, cc + b3_ref[...]], axis=1)


def _lin(x, w1, b1, w2, w3, b3):
    return pl.pallas_call(
        _lin_body,
        grid=(GR, NC),
        in_specs=[
            pl.BlockSpec((RB, D), lambda i, j: (i, 0)),
            pl.BlockSpec((D, HALF), lambda i, j: (0, j)),
            pl.BlockSpec((1, HALF), lambda i, j: (0, j)),
            pl.BlockSpec((D, HALF), lambda i, j: (0, j)),
            pl.BlockSpec((D, HALF), lambda i, j: (0, j)),
            pl.BlockSpec((1, HALF), lambda i, j: (0, j)),
        ],
        out_specs=[
            pl.BlockSpec((RB, HALF), lambda i, j: (i, j)),
            pl.BlockSpec((RB, D), lambda i, j: (i, j)),
        ],
        out_shape=[
            jax.ShapeDtypeStruct((N, D), jnp.float32),
            jax.ShapeDtypeStruct((N, 2 * D), jnp.float32),
        ],
    )(x, w1, b1.reshape(1, D), w2, w3, b3.reshape(1, D))


def _combine_body(agg_ref, deg_ref, bc_ref, h_ref):
    d = deg_ref[...]
    b = bc_ref[:, :HALF]
    cc = bc_ref[:, HALF:]
    h = agg_ref[...] - b * d + cc
    h_ref[...] = jnp.where(h >= 0, h, 0.01 * h)


def _combine(agg, bc, deg):
    return pl.pallas_call(
        _combine_body,
        grid=(GR, 2),
        in_specs=[
            pl.BlockSpec((RB, HALF), lambda i, j: (i, j)),
            pl.BlockSpec((RB, 1), lambda i, j: (i, 0)),
            pl.BlockSpec((RB, D), lambda i, j: (i, j)),
        ],
        out_specs=pl.BlockSpec((RB, HALF), lambda i, j: (i, j)),
        out_shape=jax.ShapeDtypeStruct((N, D), jnp.float32),
    )(agg, deg, bc)


def _logits_body(h_ref, w_ref, b_ref, p_ref):
    p_ref[...] = (jnp.dot(h_ref[...], w_ref[...],
                          preferred_element_type=jnp.float32) + b_ref[0, 0])


def _logits(h, pc_w_pad, pc_b):
    return pl.pallas_call(
        _logits_body,
        grid=(GR,),
        in_specs=[
            pl.BlockSpec((RB, D), lambda i: (i, 0)),
            pl.BlockSpec((D, 128), lambda i: (0, 0)),
            pl.BlockSpec((1, 1), lambda i: (0, 0)),
        ],
        out_specs=pl.BlockSpec((RB, 128), lambda i: (i, 0)),
        out_shape=jax.ShapeDtypeStruct((N, 128), jnp.float32),
    )(h, pc_w_pad, pc_b.reshape(1, 1))


def kernel(x, edge_index, edge_weight,
           W1_0, b1_0, W2_0, W3_0, b3_0,
           W1_1, b1_1, W2_1, W3_1, b3_1,
           pc_W, pc_b):
    src = edge_index[0]
    dst = edge_index[1]
    deg = jax.ops.segment_sum(edge_weight, dst,
                              num_segments=N).reshape(N, 1)

    def _layer(h, w):
        w1, b1, w2, w3, b3 = w
        a, bc = _lin(h, w1, b1, w2, w3, b3)
        agg = jax.ops.segment_sum(a[src] * edge_weight[:, None], dst,
                                  num_segments=N)
        return _combine(agg, bc, deg), None

    ws = (jnp.stack([W1_0, W1_1]), jnp.stack([b1_0, b1_1]),
          jnp.stack([W2_0, W2_1]), jnp.stack([W3_0, W3_1]),
          jnp.stack([b3_0, b3_1]))
    h, _ = lax.scan(_layer, x, ws)

    pc_w_pad = jnp.pad(pc_W, ((0, 0), (0, 127)))
    p = _logits(h, pc_w_pad, pc_b)[:, :1]
    return (h, p)
